# baseline (speedup 1.0000x reference)
import jax
import jax.numpy as jnp
from jax import lax
from jax.experimental import pallas as pl
from jax.experimental.pallas import tpu as pltpu
from jax.experimental.pallas import tpu_sc as plsc

N_NODES = 10000
N_EDGES = 320000
D_FEAT = 128
D_EDGE = 16
D_OUT = 16

NC, NS = 2, 16
NW = NC * NS
C = 512
CR = C // 16
NCHT = N_EDGES // C
KSTEPS = -(-NCHT // NW)
GSLICE = [(j * 128, 128) for j in range(C // 128)]

EB = 2560
NROWS = N_EDGES // 16


def _node_proj_body(x_ref, ws_ref, wr_ref, ps_ref, pr_ref):
    x = x_ref[...]
    ps_ref[...] = jnp.dot(x, ws_ref[...], preferred_element_type=jnp.float32)
    pr_ref[...] = jnp.dot(x, wr_ref[...], preferred_element_type=jnp.float32)


def _edge_lin_t_body(w_ref, e_ref, b_ref, o_ref):
    res = (
        jnp.dot(w_ref[...], e_ref[...], preferred_element_type=jnp.float32)
        + b_ref[...]
    )
    for s in range(2):
        for t in range(EB // 128):
            o_ref[s, t * 8:(t + 1) * 8, :] = (
                res[s * 8:(s + 1) * 8, t * 128:(t + 1) * 128])


def _sc_body(ps_hbm, pr_hbm, at_hbm, ei_hbm, out_hbm,
             sidx0, ridx0, acc0, rs0, rr0, ob0,
             sidx1, ridx1, acc1, rs1, rr1, ob1,
             sidx2, ridx2, acc2, rs2, rr2, ob2,
             semA0, semG0, semO0, semA1, semG1, semO1,
             semA2, semG2, semO2):
    wid = lax.axis_index("s") * NC + lax.axis_index("c")
    iota16 = lax.iota(jnp.int32, 16)

    slots = (
        (sidx0, ridx0, acc0, rs0, rr0, ob0, semA0, semG0, semO0),
        (sidx1, ridx1, acc1, rs1, rr1, ob1, semA1, semG1, semO1),
        (sidx2, ridx2, acc2, rs2, rr2, ob2, semA2, semG2, semO2),
    )

    def cid(n):
        return n * NW + wid

    def stage_copies(n, s):
        sidx, ridx, acc, _, _, _, semA, _, _ = slots[s]
        base = cid(n) * C
        rbase = cid(n) * CR
        return [
            (ei_hbm.at[0, pl.ds(base, C)], sidx, semA),
            (ei_hbm.at[1, pl.ds(base, C)], ridx, semA),
            (at_hbm.at[0, pl.ds(rbase, CR)], acc.at[0], semA),
            (at_hbm.at[1, pl.ds(rbase, CR)], acc.at[1], semA),
        ]

    def gather_copies(s):
        sidx, ridx, _, rs, rr, _, _, semG, _ = slots[s]
        cps = []
        for off, ln in GSLICE:
            cps.append((ps_hbm.at[sidx.at[pl.ds(off, ln)]],
                        rs.at[pl.ds(off, ln)], semG))
            cps.append((pr_hbm.at[ridx.at[pl.ds(off, ln)]],
                        rr.at[pl.ds(off, ln)], semG))
        return cps

    def out_copy(n, s):
        _, _, _, _, _, ob, _, _, semO = slots[s]
        rbase = cid(n) * CR
        return [
            (ob.at[0], out_hbm.at[0, pl.ds(rbase, CR)], semO),
            (ob.at[1], out_hbm.at[1, pl.ds(rbase, CR)], semO),
        ]

    def start(cps):
        for src, dst, sem in cps:
            pltpu.async_copy(src, dst, sem)

    def drain(cps):
        for src, dst, sem in cps:
            pltpu.make_async_copy(src, dst, sem).wait()

    def compute(s):
        _, _, acc, rs, rr, ob, _, _, _ = slots[s]

        @plsc.parallel_loop(0, C // 16, unroll=2)
        def _add_group(q):
            rows = iota16 + q * 16
            t = q // 8
            colo = q * 16 - t * 128
            for d in range(16):
                slab, r = d // 8, d % 8
                m = t * 8 + r
                cold = jnp.full((16,), d, jnp.int32)
                vs = plsc.load_gather(rs, [rows, cold])
                vr = plsc.load_gather(rr, [rows, cold])
                ob[slab, m, pl.ds(colo, 16)] = (
                    acc[slab, m, pl.ds(colo, 16)] + vs + vr)

    def step(n, s, first=False, fire_next=True, stage_next=True):
        s1 = (s + 1) % 3
        s2 = (s + 2) % 3
        if fire_next:
            @pl.when(cid(n + 1) < NCHT)
            def _():
                drain(stage_copies(n + 1, s1))
                start(gather_copies(s1))
        if stage_next:
            @pl.when(cid(n + 2) < NCHT)
            def _():
                start(stage_copies(n + 2, s2))
        drain(gather_copies(s))
        if not first:
            drain(out_copy(n - 3, s))
        compute(s)
        start(out_copy(n, s))

    start(stage_copies(0, 0))
    start(stage_copies(1, 1))
    drain(stage_copies(0, 0))
    start(gather_copies(0))

    def loop_body(m, carry):
        n = m * 3
        step(n, 0)
        step(n + 1, 1)
        step(n + 2, 2)
        return carry

    step(0, 0, first=True)
    step(1, 1, first=True)
    step(2, 2, first=True)
    lax.fori_loop(1, 6, loop_body, 0)
    step(KSTEPS - 2, (KSTEPS - 2) % 3, stage_next=False)
    drain(out_copy(KSTEPS - 4, (KSTEPS - 4) % 3))

    @pl.when(cid(KSTEPS - 1) < NCHT)
    def _():
        s = (KSTEPS - 1) % 3
        drain(gather_copies(s))
        compute(s)
        start(out_copy(KSTEPS - 1, s))
        drain(out_copy(KSTEPS - 1, s))

    drain(out_copy(KSTEPS - 3, (KSTEPS - 3) % 3))
    drain(out_copy(KSTEPS - 2, (KSTEPS - 2) % 3))


def _make_sc_gather_add():
    return pl.kernel(
        _sc_body,
        out_type=jax.ShapeDtypeStruct((2, NROWS, 128), jnp.float32),
        mesh=plsc.VectorSubcoreMesh(
            core_axis_name="c", subcore_axis_name="s",
            num_cores=NC, num_subcores=NS),
        scratch_types=(
            [
                pltpu.VMEM((C,), jnp.int32),
                pltpu.VMEM((C,), jnp.int32),
                pltpu.VMEM((2, CR, 128), jnp.float32),
                pltpu.VMEM((C, D_OUT), jnp.float32),
                pltpu.VMEM((C, D_OUT), jnp.float32),
                pltpu.VMEM((2, CR, 128), jnp.float32),
            ] * 3
            + [pltpu.SemaphoreType.DMA] * 9
        ),
        compiler_params=pltpu.CompilerParams(
            use_tc_tiling_on_sc=False, needs_layout_passes=False),
    )


def kernel(x, edge_attr, edge_index, W, b):
    ei = edge_index.astype(jnp.int32)
    we = W[:D_EDGE]
    ws = W[D_EDGE:D_EDGE + D_FEAT]
    wr = W[D_EDGE + D_FEAT:]

    ps, pr = pl.pallas_call(
        _node_proj_body,
        out_shape=[jax.ShapeDtypeStruct((N_NODES, D_OUT), jnp.float32)] * 2,
    )(x, ws, wr)

    e_t = edge_attr.T
    a4 = pl.pallas_call(
        _edge_lin_t_body,
        grid=(N_EDGES // EB,),
        in_specs=[
            pl.BlockSpec((D_EDGE, D_EDGE), lambda i: (0, 0)),
            pl.BlockSpec((D_EDGE, EB), lambda i: (0, i)),
            pl.BlockSpec((D_OUT, 1), lambda i: (0, 0)),
        ],
        out_specs=pl.BlockSpec((2, EB // 16, 128), lambda i: (0, i, 0)),
        out_shape=jax.ShapeDtypeStruct((2, NROWS, 128), jnp.float32),
    )(we.T, e_t, b[:, None])

    out4 = _make_sc_gather_add()(ps, pr, a4, ei)
    return (out4.reshape(2, N_EDGES // 128, 8, 128)
            .transpose(1, 3, 0, 2)
            .reshape(N_EDGES, D_OUT))


# speedup vs baseline: 1.2881x; 1.2881x over previous
import jax
import jax.numpy as jnp
from jax import lax
from jax.experimental import pallas as pl
from jax.experimental.pallas import tpu as pltpu
from jax.experimental.pallas import tpu_sc as plsc

N_NODES = 10000
N_EDGES = 320000
D_FEAT = 128
D_EDGE = 16
D_OUT = 16

NC, NS = 2, 16
NW = NC * NS
C = 512
CR = C // 16
NCHT = N_EDGES // C
KSTEPS = -(-NCHT // NW)
GSLICE = [(j * 128, 128) for j in range(C // 128)]

EB = 32000
NROWS = N_EDGES // 16


def _node_proj_body(x_ref, ws_ref, wr_ref, ps_ref, pr_ref):
    x = x_ref[...]
    ps_ref[...] = jnp.dot(x, ws_ref[...], preferred_element_type=jnp.float32)
    pr_ref[...] = jnp.dot(x, wr_ref[...], preferred_element_type=jnp.float32)


def _edge_lin_t_body(w_ref, e_ref, b_ref, o_ref):
    o_ref[...] = (
        jnp.dot(w_ref[...], e_ref[...], preferred_element_type=jnp.float32)
        + b_ref[...]
    )


def _sc_body(ps_hbm, pr_hbm, at_hbm, ei_hbm, out_hbm,
             sidx0, ridx0, acc0, rs0, rr0, ob0,
             sidx1, ridx1, acc1, rs1, rr1, ob1,
             sidx2, ridx2, acc2, rs2, rr2, ob2,
             semA0, semG0, semO0, semA1, semG1, semO1,
             semA2, semG2, semO2):
    wid = lax.axis_index("s") * NC + lax.axis_index("c")
    iota16 = lax.iota(jnp.int32, 16)

    slots = (
        (sidx0, ridx0, acc0, rs0, rr0, ob0, semA0, semG0, semO0),
        (sidx1, ridx1, acc1, rs1, rr1, ob1, semA1, semG1, semO1),
        (sidx2, ridx2, acc2, rs2, rr2, ob2, semA2, semG2, semO2),
    )

    def cid(n):
        return n * NW + wid

    def stage_copies(n, s):
        sidx, ridx, acc, _, _, _, semA, _, _ = slots[s]
        base = cid(n) * C
        rbase = cid(n) * CR
        del rbase
        return [
            (ei_hbm.at[0, pl.ds(base, C)], sidx, semA),
            (ei_hbm.at[1, pl.ds(base, C)], ridx, semA),
            (at_hbm.at[:, pl.ds(base, C)], acc, semA),
        ]

    def gather_copies(s):
        sidx, ridx, _, rs, rr, _, _, semG, _ = slots[s]
        cps = []
        for off, ln in GSLICE:
            cps.append((ps_hbm.at[sidx.at[pl.ds(off, ln)]],
                        rs.at[pl.ds(off, ln)], semG))
            cps.append((pr_hbm.at[ridx.at[pl.ds(off, ln)]],
                        rr.at[pl.ds(off, ln)], semG))
        return cps

    def out_copy(n, s):
        _, _, _, _, _, ob, _, _, semO = slots[s]
        rbase = cid(n) * CR
        return [
            (ob.at[0], out_hbm.at[0, pl.ds(rbase, CR)], semO),
            (ob.at[1], out_hbm.at[1, pl.ds(rbase, CR)], semO),
        ]

    def start(cps):
        for src, dst, sem in cps:
            pltpu.async_copy(src, dst, sem)

    def drain(cps):
        for src, dst, sem in cps:
            pltpu.make_async_copy(src, dst, sem).wait()

    def compute(s):
        _, _, acc, rs, rr, ob, _, _, _ = slots[s]

        @plsc.parallel_loop(0, C // 16, unroll=2)
        def _add_group(q):
            rows = iota16 + q * 16
            t = q // 8
            colo = q * 16 - t * 128
            for d in range(16):
                slab, r = d // 8, d % 8
                m = t * 8 + r
                cold = jnp.full((16,), d, jnp.int32)
                vs = plsc.load_gather(rs, [rows, cold])
                vr = plsc.load_gather(rr, [rows, cold])
                off = pl.multiple_of(q * 16, 16)
                ob[slab, m, pl.ds(colo, 16)] = (
                    acc[d, pl.ds(off, 16)] + vs + vr)

    def step(n, s, first=False, fire_next=True, stage_next=True):
        s1 = (s + 1) % 3
        s2 = (s + 2) % 3
        if fire_next:
            @pl.when(cid(n + 1) < NCHT)
            def _():
                drain(stage_copies(n + 1, s1))
                start(gather_copies(s1))
        if stage_next:
            @pl.when(cid(n + 2) < NCHT)
            def _():
                start(stage_copies(n + 2, s2))
        drain(gather_copies(s))
        if not first:
            drain(out_copy(n - 3, s))
        compute(s)
        start(out_copy(n, s))

    start(stage_copies(0, 0))
    start(stage_copies(1, 1))
    drain(stage_copies(0, 0))
    start(gather_copies(0))

    def loop_body(m, carry):
        n = m * 3
        step(n, 0)
        step(n + 1, 1)
        step(n + 2, 2)
        return carry

    step(0, 0, first=True)
    step(1, 1, first=True)
    step(2, 2, first=True)
    lax.fori_loop(1, 6, loop_body, 0)
    step(KSTEPS - 2, (KSTEPS - 2) % 3, stage_next=False)
    drain(out_copy(KSTEPS - 4, (KSTEPS - 4) % 3))

    @pl.when(cid(KSTEPS - 1) < NCHT)
    def _():
        s = (KSTEPS - 1) % 3
        drain(gather_copies(s))
        compute(s)
        start(out_copy(KSTEPS - 1, s))
        drain(out_copy(KSTEPS - 1, s))

    drain(out_copy(KSTEPS - 3, (KSTEPS - 3) % 3))
    drain(out_copy(KSTEPS - 2, (KSTEPS - 2) % 3))


def _make_sc_gather_add():
    return pl.kernel(
        _sc_body,
        out_type=jax.ShapeDtypeStruct((2, NROWS, 128), jnp.float32),
        mesh=plsc.VectorSubcoreMesh(
            core_axis_name="c", subcore_axis_name="s",
            num_cores=NC, num_subcores=NS),
        scratch_types=(
            [
                pltpu.VMEM((C,), jnp.int32),
                pltpu.VMEM((C,), jnp.int32),
                pltpu.VMEM((D_OUT, C), jnp.float32),
                pltpu.VMEM((C, D_OUT), jnp.float32),
                pltpu.VMEM((C, D_OUT), jnp.float32),
                pltpu.VMEM((2, CR, 128), jnp.float32),
            ] * 3
            + [pltpu.SemaphoreType.DMA] * 9
        ),
        compiler_params=pltpu.CompilerParams(
            use_tc_tiling_on_sc=False, needs_layout_passes=False),
    )


def kernel(x, edge_attr, edge_index, W, b):
    ei = edge_index.astype(jnp.int32)
    we = W[:D_EDGE]
    ws = W[D_EDGE:D_EDGE + D_FEAT]
    wr = W[D_EDGE + D_FEAT:]

    ps, pr = pl.pallas_call(
        _node_proj_body,
        out_shape=[jax.ShapeDtypeStruct((N_NODES, D_OUT), jnp.float32)] * 2,
    )(x, ws, wr)

    e_t = edge_attr.T
    a_t = pl.pallas_call(
        _edge_lin_t_body,
        grid=(N_EDGES // EB,),
        in_specs=[
            pl.BlockSpec((D_EDGE, D_EDGE), lambda i: (0, 0)),
            pl.BlockSpec((D_EDGE, EB), lambda i: (0, i)),
            pl.BlockSpec((D_OUT, 1), lambda i: (0, 0)),
        ],
        out_specs=pl.BlockSpec((D_OUT, EB), lambda i: (0, i)),
        out_shape=jax.ShapeDtypeStruct((D_OUT, N_EDGES), jnp.float32),
    )(we.T, e_t, b[:, None])

    out4 = _make_sc_gather_add()(ps, pr, a_t, ei)
    return (out4.reshape(2, N_EDGES // 128, 8, 128)
            .transpose(1, 3, 0, 2)
            .reshape(N_EDGES, D_OUT))


# speedup vs baseline: 1.4414x; 1.1190x over previous
import jax
import jax.numpy as jnp
from jax import lax
from jax.experimental import pallas as pl
from jax.experimental.pallas import tpu as pltpu
from jax.experimental.pallas import tpu_sc as plsc

N_NODES = 10000
N_EDGES = 320000
D_FEAT = 128
D_EDGE = 16
D_OUT = 16

NC, NS = 2, 16
NW = NC * NS
C = 512
CR = C // 16
NCHT = N_EDGES // C
KSTEPS = -(-NCHT // NW)
GSLICE = [(j * 128, 128) for j in range(C // 128)]

EB = 32000
NROWS = N_EDGES // 16


def _node_proj_body(x_ref, ws_ref, wr_ref, ps_ref, pr_ref):
    x = x_ref[...]
    ps_ref[...] = jnp.dot(x, ws_ref[...], preferred_element_type=jnp.float32)
    pr_ref[...] = jnp.dot(x, wr_ref[...], preferred_element_type=jnp.float32)


def _edge_lin_t_body(w_ref, e_ref, b_ref, o_ref):
    o_ref[...] = (
        jnp.dot(w_ref[...], e_ref[...], preferred_element_type=jnp.float32)
        + b_ref[...]
    )


def _sc_body(ps_hbm, pr_hbm, at_hbm, ei_hbm, out_hbm,
             sidx0, ridx0, acc0, rs0, rr0, ob0,
             sidx1, ridx1, acc1, rs1, rr1, ob1,
             sidx2, ridx2, acc2, rs2, rr2, ob2,
             semA0, semG0, semO0, semA1, semG1, semO1,
             semA2, semG2, semO2):
    wid = lax.axis_index("s") * NC + lax.axis_index("c")
    iota16 = lax.iota(jnp.int32, 16)

    slots = (
        (sidx0, ridx0, acc0, rs0, rr0, ob0, semA0, semG0, semO0),
        (sidx1, ridx1, acc1, rs1, rr1, ob1, semA1, semG1, semO1),
        (sidx2, ridx2, acc2, rs2, rr2, ob2, semA2, semG2, semO2),
    )

    def cid(n):
        return n * NW + wid

    def stage_copies(n, s):
        sidx, ridx, acc, _, _, _, semA, _, _ = slots[s]
        base = cid(n) * C
        rbase = cid(n) * CR
        return [
            (ei_hbm.at[0, pl.ds(base, C)], sidx, semA),
            (ei_hbm.at[1, pl.ds(base, C)], ridx, semA),
            (at_hbm.at[0, pl.ds(rbase, CR)], acc.at[0], semA),
            (at_hbm.at[1, pl.ds(rbase, CR)], acc.at[1], semA),
        ]

    def gather_copies(s):
        sidx, ridx, _, rs, rr, _, _, semG, _ = slots[s]
        cps = []
        for off, ln in GSLICE:
            cps.append((ps_hbm.at[sidx.at[pl.ds(off, ln)]],
                        rs.at[pl.ds(off, ln)], semG))
            cps.append((pr_hbm.at[ridx.at[pl.ds(off, ln)]],
                        rr.at[pl.ds(off, ln)], semG))
        return cps

    def out_copy(n, s):
        _, _, _, _, _, ob, _, _, semO = slots[s]
        rbase = cid(n) * CR
        return [
            (ob.at[0], out_hbm.at[0, pl.ds(rbase, CR)], semO),
            (ob.at[1], out_hbm.at[1, pl.ds(rbase, CR)], semO),
        ]

    def start(cps):
        for src, dst, sem in cps:
            pltpu.async_copy(src, dst, sem)

    def drain(cps):
        for src, dst, sem in cps:
            pltpu.make_async_copy(src, dst, sem).wait()

    def compute(s):
        _, _, acc, rs, rr, ob, _, _, _ = slots[s]

        @plsc.parallel_loop(0, C // 16, unroll=2)
        def _add_group(q):
            rows = iota16 + q * 16
            t = q // 8
            colo = q * 16 - t * 128
            for d in range(16):
                slab, r = d // 8, d % 8
                m = t * 8 + r
                cold = jnp.full((16,), d, jnp.int32)
                vs = plsc.load_gather(rs, [rows, cold])
                vr = plsc.load_gather(rr, [rows, cold])
                ob[slab, m, pl.ds(colo, 16)] = (
                    acc[slab, m, pl.ds(colo, 16)] + vs + vr)

    def step(n, s, first=False, fire_next=True, stage_next=True):
        s1 = (s + 1) % 3
        s2 = (s + 2) % 3
        if fire_next:
            @pl.when(cid(n + 1) < NCHT)
            def _():
                drain(stage_copies(n + 1, s1))
                start(gather_copies(s1))
        if stage_next:
            @pl.when(cid(n + 2) < NCHT)
            def _():
                start(stage_copies(n + 2, s2))
        drain(gather_copies(s))
        if not first:
            drain(out_copy(n - 3, s))
        compute(s)
        start(out_copy(n, s))

    start(stage_copies(0, 0))
    start(stage_copies(1, 1))
    drain(stage_copies(0, 0))
    start(gather_copies(0))

    def loop_body(m, carry):
        n = m * 3
        step(n, 0)
        step(n + 1, 1)
        step(n + 2, 2)
        return carry

    step(0, 0, first=True)
    step(1, 1, first=True)
    step(2, 2, first=True)
    lax.fori_loop(1, 6, loop_body, 0)
    step(KSTEPS - 2, (KSTEPS - 2) % 3, stage_next=False)
    drain(out_copy(KSTEPS - 4, (KSTEPS - 4) % 3))

    @pl.when(cid(KSTEPS - 1) < NCHT)
    def _():
        s = (KSTEPS - 1) % 3
        drain(gather_copies(s))
        compute(s)
        start(out_copy(KSTEPS - 1, s))
        drain(out_copy(KSTEPS - 1, s))

    drain(out_copy(KSTEPS - 3, (KSTEPS - 3) % 3))
    drain(out_copy(KSTEPS - 2, (KSTEPS - 2) % 3))


def _make_sc_gather_add():
    return pl.kernel(
        _sc_body,
        out_type=jax.ShapeDtypeStruct((2, NROWS, 128), jnp.float32),
        mesh=plsc.VectorSubcoreMesh(
            core_axis_name="c", subcore_axis_name="s",
            num_cores=NC, num_subcores=NS),
        scratch_types=(
            [
                pltpu.VMEM((C,), jnp.int32),
                pltpu.VMEM((C,), jnp.int32),
                pltpu.VMEM((2, CR, 128), jnp.float32),
                pltpu.VMEM((C, D_OUT), jnp.float32),
                pltpu.VMEM((C, D_OUT), jnp.float32),
                pltpu.VMEM((2, CR, 128), jnp.float32),
            ] * 3
            + [pltpu.SemaphoreType.DMA] * 9
        ),
        compiler_params=pltpu.CompilerParams(
            use_tc_tiling_on_sc=False, needs_layout_passes=False),
    )


def kernel(x, edge_attr, edge_index, W, b):
    ei = edge_index.astype(jnp.int32)
    we = W[:D_EDGE]
    ws = W[D_EDGE:D_EDGE + D_FEAT]
    wr = W[D_EDGE + D_FEAT:]

    ps, pr = pl.pallas_call(
        _node_proj_body,
        out_shape=[jax.ShapeDtypeStruct((N_NODES, D_OUT), jnp.float32)] * 2,
    )(x, ws, wr)

    e_t = edge_attr.T
    a_t = pl.pallas_call(
        _edge_lin_t_body,
        grid=(N_EDGES // EB,),
        in_specs=[
            pl.BlockSpec((D_EDGE, D_EDGE), lambda i: (0, 0)),
            pl.BlockSpec((D_EDGE, EB), lambda i: (0, i)),
            pl.BlockSpec((D_OUT, 1), lambda i: (0, 0)),
        ],
        out_specs=pl.BlockSpec((D_OUT, EB), lambda i: (0, i)),
        out_shape=jax.ShapeDtypeStruct((D_OUT, N_EDGES), jnp.float32),
    )(we.T, e_t, b[:, None])

    a4 = (a_t.reshape(2, 8, N_EDGES // 128, 128).transpose(0, 2, 1, 3)
          .reshape(2, NROWS, 128))
    out4 = _make_sc_gather_add()(ps, pr, a4, ei)
    return (out4.reshape(2, N_EDGES // 128, 8, 128)
            .transpose(1, 3, 0, 2)
            .reshape(N_EDGES, D_OUT))
